# baseline (speedup 1.0000x reference)
import functools

import jax
import jax.numpy as jnp
from jax import lax
from jax.experimental import pallas as pl
from jax.experimental.pallas import tpu as pltpu
from jax.experimental.pallas import tpu_sc as plsc

_BT = 512


def _cbw_body(cb_ref, w_ref, o_ref):
    o_ref[...] = lax.dot_general(cb_ref[...], w_ref[...],
                                 (((1,), (1,)), ((), ())),
                                 preferred_element_type=jnp.float32)


def _s1_body(h_ref, w_ref, cb_ref, idx_ref, acc_ref):
    i = pl.program_id(0)
    cb = cb_ref[...]
    z = lax.dot_general(h_ref[...].astype(jnp.bfloat16), w_ref[...],
                        (((1,), (1,)), ((), ())),
                        preferred_element_type=jnp.float32)
    s = lax.dot_general(z.astype(jnp.bfloat16), cb,
                        (((1,), (1,)), ((), ())),
                        preferred_element_type=jnp.float32)
    cbf = cb.astype(jnp.float32)
    ones = jnp.ones((1, cb.shape[1]), jnp.float32)
    cnorm = lax.dot_general(ones, cbf * cbf, (((1,), (1,)), ((), ())),
                            preferred_element_type=jnp.float32)
    d = cnorm - 2.0 * s
    dmin = jnp.min(d, axis=1, keepdims=True)
    cols = lax.broadcasted_iota(jnp.int32, d.shape, 1)
    idx_ref[...] = jnp.min(jnp.where(d <= dmin, cols, d.shape[1]),
                           axis=1, keepdims=True)
    znorm = jnp.sum(z * z, axis=1, keepdims=True)
    part = jnp.sum(znorm + dmin, axis=(0, 1), keepdims=True)

    @pl.when(i == 0)
    def _init():
        acc_ref[...] = part

    @pl.when(i != 0)
    def _accum():
        acc_ref[...] += part


def _s3_body(out_ref, h_ref, a_ref, g_ref, b_ref, o_ref):
    x = h_ref[...] + a_ref[...] * out_ref[...]
    mu = jnp.mean(x, axis=1, keepdims=True)
    xc = x - mu
    var = jnp.mean(xc * xc, axis=1, keepdims=True)
    o_ref[...] = xc * lax.rsqrt(var + 1e-5) * g_ref[...] + b_ref[...]


def _make_sc_gather(num_tokens, dim):
    info = plsc.get_sparse_core_info()
    nc, ns = info.num_cores, info.num_subcores
    nw = nc * ns
    b_per_w = num_tokens // nw
    ch = 64
    n_ch = b_per_w // ch
    mesh = plsc.VectorSubcoreMesh(core_axis_name="c", subcore_axis_name="s")

    @functools.partial(
        pl.kernel, mesh=mesh,
        out_type=jax.ShapeDtypeStruct((num_tokens, dim), jnp.float32),
        scratch_types=[
            pltpu.VMEM((ch,), jnp.int32),
            pltpu.VMEM((ch, dim), jnp.float32),
            pltpu.SemaphoreType.DMA,
        ],
    )
    def gather(table_hbm, idx_hbm, out_hbm, idx_v, rows_v, sem):
        wid = lax.axis_index("s") * nc + lax.axis_index("c")
        base = wid * b_per_w
        for c in range(n_ch):
            off = base + c * ch
            pltpu.sync_copy(idx_hbm.at[pl.ds(off, ch)], idx_v)
            pltpu.async_copy(table_hbm.at[idx_v], rows_v, sem).wait()
            pltpu.sync_copy(rows_v, out_hbm.at[pl.ds(off, ch)])

    return gather


def kernel(hidden, codebook, W_in, W_out, ln_g, ln_b, active_mask):
    d = hidden.shape[-1]
    n = hidden.shape[0] * hidden.shape[1]
    c = codebook.shape[0]
    h2 = hidden.reshape(n, d)
    nblk = n // _BT

    cbw = pl.pallas_call(
        _cbw_body,
        grid=(1,),
        in_specs=[
            pl.BlockSpec((c, d), lambda i: (0, 0)),
            pl.BlockSpec((d, d), lambda i: (0, 0)),
        ],
        out_specs=pl.BlockSpec((c, d), lambda i: (0, 0)),
        out_shape=jax.ShapeDtypeStruct((c, d), jnp.float32),
    )(codebook, W_out)

    idx, acc = pl.pallas_call(
        _s1_body,
        grid=(nblk,),
        in_specs=[
            pl.BlockSpec((_BT, d), lambda i: (i, 0)),
            pl.BlockSpec((d, d), lambda i: (0, 0)),
            pl.BlockSpec((c, d), lambda i: (0, 0)),
        ],
        out_specs=[
            pl.BlockSpec((_BT, 1), lambda i: (i, 0)),
            pl.BlockSpec((1, 1), lambda i: (0, 0)),
        ],
        out_shape=[
            jax.ShapeDtypeStruct((n, 1), jnp.int32),
            jax.ShapeDtypeStruct((1, 1), jnp.float32),
        ],
    )(h2, W_in.astype(jnp.bfloat16), codebook.astype(jnp.bfloat16))

    out_rows = _make_sc_gather(n, d)(cbw, idx.reshape(n))

    active_f = active_mask.reshape(n, 1).astype(jnp.float32)
    h_comm = pl.pallas_call(
        _s3_body,
        grid=(nblk,),
        in_specs=[
            pl.BlockSpec((_BT, d), lambda i: (i, 0)),
            pl.BlockSpec((_BT, d), lambda i: (i, 0)),
            pl.BlockSpec((_BT, 1), lambda i: (i, 0)),
            pl.BlockSpec((1, d), lambda i: (0, 0)),
            pl.BlockSpec((1, d), lambda i: (0, 0)),
        ],
        out_specs=pl.BlockSpec((_BT, d), lambda i: (i, 0)),
        out_shape=jax.ShapeDtypeStruct((n, d), jnp.float32),
    )(out_rows, h2, active_f, ln_g.reshape(1, d), ln_b.reshape(1, d))

    vq_loss = (1.0 + 0.25) * acc[0, 0] / (n * d)
    return h_comm.reshape(hidden.shape), vq_loss


# speedup vs baseline: 1.1083x; 1.1083x over previous
import functools

import jax
import jax.numpy as jnp
from jax import lax
from jax.experimental import pallas as pl
from jax.experimental.pallas import tpu as pltpu
from jax.experimental.pallas import tpu_sc as plsc

_BT = 512


def _rtne_bf16_bits(u):
    return u + jnp.uint32(0x7FFF) + ((u >> 16) & jnp.uint32(1))


def _cbw_body(cb_ref, w_ref, o_ref):
    m = lax.dot_general(cb_ref[...], w_ref[...], (((1,), (1,)), ((), ())),
                        preferred_element_type=jnp.float32)
    half = m.shape[1] // 2
    u_lo = _rtne_bf16_bits(lax.bitcast_convert_type(m[:, :half], jnp.uint32))
    u_hi = _rtne_bf16_bits(lax.bitcast_convert_type(m[:, half:], jnp.uint32))
    packed = (u_hi & jnp.uint32(0xFFFF0000)) | (u_lo >> 16)
    o_ref[...] = lax.bitcast_convert_type(packed, jnp.int32)


def _s1_body(h_ref, w_ref, cb_ref, idx_ref, acc_ref):
    i = pl.program_id(0)
    cb = cb_ref[...]
    z = lax.dot_general(h_ref[...].astype(jnp.bfloat16), w_ref[...],
                        (((1,), (1,)), ((), ())),
                        preferred_element_type=jnp.float32)
    s = lax.dot_general(z.astype(jnp.bfloat16), cb,
                        (((1,), (1,)), ((), ())),
                        preferred_element_type=jnp.float32)
    cbf = cb.astype(jnp.float32)
    ones = jnp.ones((1, cb.shape[1]), jnp.float32)
    cnorm = lax.dot_general(ones, cbf * cbf, (((1,), (1,)), ((), ())),
                            preferred_element_type=jnp.float32)
    d = cnorm - 2.0 * s
    dmin = jnp.min(d, axis=1, keepdims=True)
    cols = lax.broadcasted_iota(jnp.int32, d.shape, 1)
    idx_ref[...] = jnp.min(jnp.where(d <= dmin, cols, d.shape[1]),
                           axis=1, keepdims=True)
    znorm = jnp.sum(z * z, axis=1, keepdims=True)
    part = jnp.sum(znorm + dmin, axis=(0, 1), keepdims=True)

    @pl.when(i == 0)
    def _init():
        acc_ref[...] = part

    @pl.when(i != 0)
    def _accum():
        acc_ref[...] += part


def _s3_body(out_ref, h_ref, a_ref, g_ref, b_ref, o_ref):
    u = lax.bitcast_convert_type(out_ref[...], jnp.uint32)
    f_lo = lax.bitcast_convert_type(u << 16, jnp.float32)
    f_hi = lax.bitcast_convert_type(u & jnp.uint32(0xFFFF0000), jnp.float32)
    half = u.shape[1]
    dim = 2 * half
    a = a_ref[...]
    x0 = h_ref[:, :half] + a * f_lo
    x1 = h_ref[:, half:] + a * f_hi
    mu = (jnp.sum(x0, axis=1, keepdims=True)
          + jnp.sum(x1, axis=1, keepdims=True)) * (1.0 / dim)
    xc0 = x0 - mu
    xc1 = x1 - mu
    var = (jnp.sum(xc0 * xc0, axis=1, keepdims=True)
           + jnp.sum(xc1 * xc1, axis=1, keepdims=True)) * (1.0 / dim)
    r = lax.rsqrt(var + 1e-5)
    o_ref[:, :half] = xc0 * r * g_ref[:, :half] + b_ref[:, :half]
    o_ref[:, half:] = xc1 * r * g_ref[:, half:] + b_ref[:, half:]


def _make_sc_gather(num_tokens, dim):
    info = plsc.get_sparse_core_info()
    nc, ns = info.num_cores, info.num_subcores
    nw = nc * ns
    b_per_w = num_tokens // nw
    ch = 128
    n_ch = b_per_w // ch
    mesh = plsc.VectorSubcoreMesh(core_axis_name="c", subcore_axis_name="s")

    @functools.partial(
        pl.kernel, mesh=mesh,
        out_type=jax.ShapeDtypeStruct((num_tokens, dim), jnp.int32),
        scratch_types=[
            pltpu.VMEM((ch,), jnp.int32),
            pltpu.VMEM((ch, dim), jnp.int32),
            pltpu.SemaphoreType.DMA,
        ],
    )
    def gather(table_hbm, idx_hbm, out_hbm, idx_v, rows_v, sem):
        wid = lax.axis_index("s") * nc + lax.axis_index("c")
        base = wid * b_per_w
        for c in range(n_ch):
            off = base + c * ch
            pltpu.sync_copy(idx_hbm.at[pl.ds(off, ch)], idx_v)
            pltpu.async_copy(table_hbm.at[idx_v], rows_v, sem).wait()
            pltpu.sync_copy(rows_v, out_hbm.at[pl.ds(off, ch)])

    return gather


def kernel(hidden, codebook, W_in, W_out, ln_g, ln_b, active_mask):
    d = hidden.shape[-1]
    n = hidden.shape[0] * hidden.shape[1]
    c = codebook.shape[0]
    h2 = hidden.reshape(n, d)
    nblk = n // _BT

    cbw = pl.pallas_call(
        _cbw_body,
        grid=(1,),
        in_specs=[
            pl.BlockSpec((c, d), lambda i: (0, 0)),
            pl.BlockSpec((d, d), lambda i: (0, 0)),
        ],
        out_specs=pl.BlockSpec((c, d // 2), lambda i: (0, 0)),
        out_shape=jax.ShapeDtypeStruct((c, d // 2), jnp.int32),
    )(codebook, W_out)

    idx, acc = pl.pallas_call(
        _s1_body,
        grid=(nblk,),
        in_specs=[
            pl.BlockSpec((_BT, d), lambda i: (i, 0)),
            pl.BlockSpec((d, d), lambda i: (0, 0)),
            pl.BlockSpec((c, d), lambda i: (0, 0)),
        ],
        out_specs=[
            pl.BlockSpec((_BT, 1), lambda i: (i, 0)),
            pl.BlockSpec((1, 1), lambda i: (0, 0)),
        ],
        out_shape=[
            jax.ShapeDtypeStruct((n, 1), jnp.int32),
            jax.ShapeDtypeStruct((1, 1), jnp.float32),
        ],
    )(h2, W_in.astype(jnp.bfloat16), codebook.astype(jnp.bfloat16))

    out_rows = _make_sc_gather(n, d // 2)(cbw, idx.reshape(n))

    active_f = active_mask.reshape(n, 1).astype(jnp.float32)
    h_comm = pl.pallas_call(
        _s3_body,
        grid=(nblk,),
        in_specs=[
            pl.BlockSpec((_BT, d // 2), lambda i: (i, 0)),
            pl.BlockSpec((_BT, d), lambda i: (i, 0)),
            pl.BlockSpec((_BT, 1), lambda i: (i, 0)),
            pl.BlockSpec((1, d), lambda i: (0, 0)),
            pl.BlockSpec((1, d), lambda i: (0, 0)),
        ],
        out_specs=pl.BlockSpec((_BT, d), lambda i: (i, 0)),
        out_shape=jax.ShapeDtypeStruct((n, d), jnp.float32),
    )(out_rows, h2, active_f, ln_g.reshape(1, d), ln_b.reshape(1, d))

    vq_loss = (1.0 + 0.25) * acc[0, 0] / (n * d)
    return h_comm.reshape(hidden.shape), vq_loss


# speedup vs baseline: 1.1129x; 1.0042x over previous
import functools

import jax
import jax.numpy as jnp
from jax import lax
from jax.experimental import pallas as pl
from jax.experimental.pallas import tpu as pltpu
from jax.experimental.pallas import tpu_sc as plsc

_BT = 512


def _rtne_bf16_bits(u):
    return u + jnp.uint32(0x7FFF) + ((u >> 16) & jnp.uint32(1))


def _cbw_body(cb_ref, w_ref, o_ref):
    m = lax.dot_general(cb_ref[...], w_ref[...], (((1,), (1,)), ((), ())),
                        preferred_element_type=jnp.float32)
    half = m.shape[1] // 2
    u_lo = _rtne_bf16_bits(lax.bitcast_convert_type(m[:, :half], jnp.uint32))
    u_hi = _rtne_bf16_bits(lax.bitcast_convert_type(m[:, half:], jnp.uint32))
    packed = (u_hi & jnp.uint32(0xFFFF0000)) | (u_lo >> 16)
    o_ref[...] = lax.bitcast_convert_type(packed, jnp.int32)


def _s1_body(h_ref, w_ref, cb_ref, idx_ref, acc_ref):
    i = pl.program_id(0)
    cb = cb_ref[...]
    z = lax.dot_general(h_ref[...].astype(jnp.bfloat16), w_ref[...],
                        (((1,), (1,)), ((), ())),
                        preferred_element_type=jnp.float32)
    s = lax.dot_general(z.astype(jnp.bfloat16), cb,
                        (((1,), (1,)), ((), ())),
                        preferred_element_type=jnp.float32)
    cbf = cb.astype(jnp.float32)
    ones = jnp.ones((1, cb.shape[1]), jnp.float32)
    cnorm = lax.dot_general(ones, cbf * cbf, (((1,), (1,)), ((), ())),
                            preferred_element_type=jnp.float32)
    d = cnorm - 2.0 * s
    dmin = jnp.min(d, axis=1, keepdims=True)
    cols = lax.broadcasted_iota(jnp.int32, d.shape, 1)
    idx_ref[...] = jnp.min(jnp.where(d <= dmin, cols, d.shape[1]),
                           axis=1, keepdims=True)
    znorm = jnp.sum(z * z, axis=1, keepdims=True)
    part = jnp.sum(znorm + dmin, axis=(0, 1), keepdims=True)

    @pl.when(i == 0)
    def _init():
        acc_ref[...] = part

    @pl.when(i != 0)
    def _accum():
        acc_ref[...] += part


def _s3_alias_body(prev_ref, out_ref, h_ref, a_ref, g_ref, b_ref, o_ref):
    del prev_ref
    _s3_body(out_ref, h_ref, a_ref, g_ref, b_ref, o_ref)


def _s3_body(out_ref, h_ref, a_ref, g_ref, b_ref, o_ref):
    u = lax.bitcast_convert_type(out_ref[...], jnp.uint32)
    f_lo = lax.bitcast_convert_type(u << 16, jnp.float32)
    f_hi = lax.bitcast_convert_type(u & jnp.uint32(0xFFFF0000), jnp.float32)
    half = u.shape[1]
    dim = 2 * half
    a = a_ref[...]
    x0 = h_ref[:, :half] + a * f_lo
    x1 = h_ref[:, half:] + a * f_hi
    mu = (jnp.sum(x0, axis=1, keepdims=True)
          + jnp.sum(x1, axis=1, keepdims=True)) * (1.0 / dim)
    xc0 = x0 - mu
    xc1 = x1 - mu
    var = (jnp.sum(xc0 * xc0, axis=1, keepdims=True)
           + jnp.sum(xc1 * xc1, axis=1, keepdims=True)) * (1.0 / dim)
    r = lax.rsqrt(var + 1e-5)
    o_ref[:, :half] = xc0 * r * g_ref[:, :half] + b_ref[:, :half]
    o_ref[:, half:] = xc1 * r * g_ref[:, half:] + b_ref[:, half:]


def _make_sc_gather(num_tokens, dim):
    info = plsc.get_sparse_core_info()
    nc, ns = info.num_cores, info.num_subcores
    nw = nc * ns
    b_per_w = num_tokens // nw
    ch = 128
    n_ch = b_per_w // ch
    mesh = plsc.VectorSubcoreMesh(core_axis_name="c", subcore_axis_name="s")

    @functools.partial(
        pl.kernel, mesh=mesh,
        out_type=jax.ShapeDtypeStruct((num_tokens, dim), jnp.int32),
        scratch_types=[
            pltpu.VMEM((ch,), jnp.int32),
            pltpu.VMEM((ch, dim), jnp.int32),
            pltpu.SemaphoreType.DMA,
        ],
    )
    def gather(table_hbm, idx_hbm, out_hbm, idx_v, rows_v, sem):
        wid = lax.axis_index("s") * nc + lax.axis_index("c")
        base = wid * b_per_w
        for c in range(n_ch):
            off = base + c * ch
            pltpu.sync_copy(idx_hbm.at[pl.ds(off, ch)], idx_v)
            pltpu.async_copy(table_hbm.at[idx_v], rows_v, sem).wait()
            pltpu.sync_copy(rows_v, out_hbm.at[pl.ds(off, ch)])

    return gather


def kernel(hidden, codebook, W_in, W_out, ln_g, ln_b, active_mask):
    d = hidden.shape[-1]
    n = hidden.shape[0] * hidden.shape[1]
    c = codebook.shape[0]
    h2 = hidden.reshape(n, d)
    nblk = n // _BT

    cbw = pl.pallas_call(
        _cbw_body,
        grid=(1,),
        in_specs=[
            pl.BlockSpec((c, d), lambda i: (0, 0)),
            pl.BlockSpec((d, d), lambda i: (0, 0)),
        ],
        out_specs=pl.BlockSpec((c, d // 2), lambda i: (0, 0)),
        out_shape=jax.ShapeDtypeStruct((c, d // 2), jnp.int32),
    )(codebook, W_out)

    w_in_bf = W_in.astype(jnp.bfloat16)
    cb_bf = codebook.astype(jnp.bfloat16)
    active_f = active_mask.reshape(n, 1).astype(jnp.float32)
    nh = n // 2
    nblk = nh // _BT
    gather = _make_sc_gather(nh, d // 2)

    def s1(p):
        return pl.pallas_call(
            _s1_body,
            grid=(nblk,),
            in_specs=[
                pl.BlockSpec((_BT, d), lambda i: (i + p * nblk, 0)),
                pl.BlockSpec((d, d), lambda i: (0, 0)),
                pl.BlockSpec((c, d), lambda i: (0, 0)),
            ],
            out_specs=[
                pl.BlockSpec((_BT, 1), lambda i: (i, 0)),
                pl.BlockSpec((1, 1), lambda i: (0, 0)),
            ],
            out_shape=[
                jax.ShapeDtypeStruct((nh, 1), jnp.int32),
                jax.ShapeDtypeStruct((1, 1), jnp.float32),
            ],
        )(h2, w_in_bf, cb_bf)

    def s3_first(rows_half):
        return pl.pallas_call(
            _s3_body,
            grid=(nblk,),
            in_specs=[
                pl.BlockSpec((_BT, d // 2), lambda i: (i, 0)),
                pl.BlockSpec((_BT, d), lambda i: (i, 0)),
                pl.BlockSpec((_BT, 1), lambda i: (i, 0)),
                pl.BlockSpec((1, d), lambda i: (0, 0)),
                pl.BlockSpec((1, d), lambda i: (0, 0)),
            ],
            out_specs=pl.BlockSpec((_BT, d), lambda i: (i, 0)),
            out_shape=jax.ShapeDtypeStruct((n, d), jnp.float32),
        )(rows_half, h2, active_f, ln_g.reshape(1, d), ln_b.reshape(1, d))

    def s3_second(prev, rows_half):
        return pl.pallas_call(
            _s3_alias_body,
            grid=(nblk,),
            in_specs=[
                pl.BlockSpec((8, 128), lambda i: (0, 0)),
                pl.BlockSpec((_BT, d // 2), lambda i: (i, 0)),
                pl.BlockSpec((_BT, d), lambda i: (i + nblk, 0)),
                pl.BlockSpec((_BT, 1), lambda i: (i + nblk, 0)),
                pl.BlockSpec((1, d), lambda i: (0, 0)),
                pl.BlockSpec((1, d), lambda i: (0, 0)),
            ],
            out_specs=pl.BlockSpec((_BT, d), lambda i: (i + nblk, 0)),
            out_shape=jax.ShapeDtypeStruct((n, d), jnp.float32),
            input_output_aliases={0: 0},
        )(prev, rows_half, h2, active_f, ln_g.reshape(1, d),
          ln_b.reshape(1, d))

    idx_a, acc_a = s1(0)
    rows_a = gather(cbw, idx_a.reshape(nh))
    idx_b, acc_b = s1(1)
    rows_b = gather(cbw, idx_b.reshape(nh))
    hc_a = s3_first(rows_a)
    h_comm = s3_second(hc_a, rows_b)

    vq_loss = (1.0 + 0.25) * (acc_a[0, 0] + acc_b[0, 0]) / (n * d)
    return h_comm.reshape(hidden.shape), vq_loss


# speedup vs baseline: 1.1853x; 1.0651x over previous
import functools

import jax
import jax.numpy as jnp
from jax import lax
from jax.experimental import pallas as pl
from jax.experimental.pallas import tpu as pltpu
from jax.experimental.pallas import tpu_sc as plsc

_BT = 512


def _rtne_bf16_bits(u):
    return u + jnp.uint32(0x7FFF) + ((u >> 16) & jnp.uint32(1))


def _prep_body(cb_ref, wi_ref, wo_ref, cbw_ref, cbbf_ref, wibf_ref):
    cb = cb_ref[...]
    m = lax.dot_general(cb, wo_ref[...], (((1,), (1,)), ((), ())),
                        preferred_element_type=jnp.float32)
    half = m.shape[1] // 2
    u_lo = _rtne_bf16_bits(lax.bitcast_convert_type(m[:, :half], jnp.uint32))
    u_hi = _rtne_bf16_bits(lax.bitcast_convert_type(m[:, half:], jnp.uint32))
    packed = (u_hi & jnp.uint32(0xFFFF0000)) | (u_lo >> 16)
    cbw_ref[...] = lax.bitcast_convert_type(packed, jnp.int32)
    cbbf_ref[...] = cb.astype(jnp.bfloat16)
    wibf_ref[...] = wi_ref[...].astype(jnp.bfloat16)


_TS = 256


def _s1_body(h_ref, w_ref, cb_ref, idx_ref, acc_ref):
    i = pl.program_id(0)
    cb = cb_ref[...]
    cbf = cb.astype(jnp.float32)
    ones = jnp.ones((1, cb.shape[1]), jnp.float32)
    cnorm = lax.dot_general(ones, cbf * cbf, (((1,), (1,)), ((), ())),
                            preferred_element_type=jnp.float32)
    part = None
    for t in range(_BT // _TS):
        h = h_ref[t * _TS:(t + 1) * _TS, :]
        z = lax.dot_general(h.astype(jnp.bfloat16), w_ref[...],
                            (((1,), (1,)), ((), ())),
                            preferred_element_type=jnp.float32)
        s = lax.dot_general(z.astype(jnp.bfloat16), cb,
                            (((1,), (1,)), ((), ())),
                            preferred_element_type=jnp.float32)
        d = cnorm - 2.0 * s
        dmin = jnp.min(d, axis=1, keepdims=True)
        cols = lax.broadcasted_iota(jnp.int32, d.shape, 1)
        idx_ref[pl.ds(t * _TS, _TS)] = jnp.min(
            jnp.where(d <= dmin, cols, d.shape[1]), axis=1)
        znorm = jnp.sum(z * z, axis=1, keepdims=True)
        p = jnp.sum(znorm + dmin, axis=(0, 1), keepdims=True)
        part = p if part is None else part + p

    @pl.when(i == 0)
    def _init():
        acc_ref[...] = part

    @pl.when(i != 0)
    def _accum():
        acc_ref[...] += part


def _s3_alias_body(prev_ref, out_ref, h_ref, a_ref, g_ref, b_ref, o_ref):
    del prev_ref
    _s3_body(out_ref, h_ref, a_ref, g_ref, b_ref, o_ref)


def _s3_body(out_ref, h_ref, a_ref, g_ref, b_ref, o_ref):
    u = lax.bitcast_convert_type(out_ref[...], jnp.uint32)
    f_lo = lax.bitcast_convert_type(u << 16, jnp.float32)
    f_hi = lax.bitcast_convert_type(u & jnp.uint32(0xFFFF0000), jnp.float32)
    half = u.shape[1]
    dim = 2 * half
    a = a_ref[...]
    x0 = h_ref[:, :half] + a * f_lo
    x1 = h_ref[:, half:] + a * f_hi
    mu = (jnp.sum(x0, axis=1, keepdims=True)
          + jnp.sum(x1, axis=1, keepdims=True)) * (1.0 / dim)
    xc0 = x0 - mu
    xc1 = x1 - mu
    var = (jnp.sum(xc0 * xc0, axis=1, keepdims=True)
           + jnp.sum(xc1 * xc1, axis=1, keepdims=True)) * (1.0 / dim)
    r = lax.rsqrt(var + 1e-5)
    o_ref[:, :half] = xc0 * r * g_ref[:, :half] + b_ref[:, :half]
    o_ref[:, half:] = xc1 * r * g_ref[:, half:] + b_ref[:, half:]


def _make_sc_gather(num_tokens, dim):
    info = plsc.get_sparse_core_info()
    nc, ns = info.num_cores, info.num_subcores
    nw = nc * ns
    b_per_w = num_tokens // nw
    ch = 128
    n_ch = b_per_w // ch
    mesh = plsc.VectorSubcoreMesh(core_axis_name="c", subcore_axis_name="s")

    @functools.partial(
        pl.kernel, mesh=mesh,
        out_type=jax.ShapeDtypeStruct((num_tokens, dim), jnp.int32),
        scratch_types=[
            pltpu.VMEM((ch,), jnp.int32),
            pltpu.VMEM((ch, dim), jnp.int32),
            pltpu.SemaphoreType.DMA,
        ],
    )
    def gather(table_hbm, idx_hbm, out_hbm, idx_v, rows_v, sem):
        wid = lax.axis_index("s") * nc + lax.axis_index("c")
        base = wid * b_per_w
        for c in range(n_ch):
            off = base + c * ch
            pltpu.sync_copy(idx_hbm.at[pl.ds(off, ch)], idx_v)
            pltpu.async_copy(table_hbm.at[idx_v], rows_v, sem).wait()
            pltpu.sync_copy(rows_v, out_hbm.at[pl.ds(off, ch)])

    return gather


def kernel(hidden, codebook, W_in, W_out, ln_g, ln_b, active_mask):
    d = hidden.shape[-1]
    n = hidden.shape[0] * hidden.shape[1]
    c = codebook.shape[0]
    h2 = hidden.reshape(n, d)
    nblk = n // _BT

    cbw, cb_bf, w_in_bf = pl.pallas_call(
        _prep_body,
        grid=(1,),
        in_specs=[
            pl.BlockSpec((c, d), lambda i: (0, 0)),
            pl.BlockSpec((d, d), lambda i: (0, 0)),
            pl.BlockSpec((d, d), lambda i: (0, 0)),
        ],
        out_specs=[
            pl.BlockSpec((c, d // 2), lambda i: (0, 0)),
            pl.BlockSpec((c, d), lambda i: (0, 0)),
            pl.BlockSpec((d, d), lambda i: (0, 0)),
        ],
        out_shape=[
            jax.ShapeDtypeStruct((c, d // 2), jnp.int32),
            jax.ShapeDtypeStruct((c, d), jnp.bfloat16),
            jax.ShapeDtypeStruct((d, d), jnp.bfloat16),
        ],
    )(codebook, W_in, W_out)

    active_f = active_mask.reshape(n, 1).astype(jnp.float32)
    nh = n // 2
    nblk = nh // _BT
    gather = _make_sc_gather(nh, d // 2)

    def s1(p):
        return pl.pallas_call(
            _s1_body,
            grid=(nblk,),
            in_specs=[
                pl.BlockSpec((_BT, d), lambda i: (i + p * nblk, 0)),
                pl.BlockSpec((d, d), lambda i: (0, 0)),
                pl.BlockSpec((c, d), lambda i: (0, 0)),
            ],
            out_specs=[
                pl.BlockSpec((_BT,), lambda i: (i,)),
                pl.BlockSpec((1, 1), lambda i: (0, 0)),
            ],
            out_shape=[
                jax.ShapeDtypeStruct((nh,), jnp.int32),
                jax.ShapeDtypeStruct((1, 1), jnp.float32),
            ],
        )(h2, w_in_bf, cb_bf)

    def s3_first(rows_half):
        return pl.pallas_call(
            _s3_body,
            grid=(nblk,),
            in_specs=[
                pl.BlockSpec((_BT, d // 2), lambda i: (i, 0)),
                pl.BlockSpec((_BT, d), lambda i: (i, 0)),
                pl.BlockSpec((_BT, 1), lambda i: (i, 0)),
                pl.BlockSpec((1, d), lambda i: (0, 0)),
                pl.BlockSpec((1, d), lambda i: (0, 0)),
            ],
            out_specs=pl.BlockSpec((_BT, d), lambda i: (i, 0)),
            out_shape=jax.ShapeDtypeStruct((n, d), jnp.float32),
        )(rows_half, h2, active_f, ln_g.reshape(1, d), ln_b.reshape(1, d))

    def s3_second(prev, rows_half):
        return pl.pallas_call(
            _s3_alias_body,
            grid=(nblk,),
            in_specs=[
                pl.BlockSpec((8, 128), lambda i: (0, 0)),
                pl.BlockSpec((_BT, d // 2), lambda i: (i, 0)),
                pl.BlockSpec((_BT, d), lambda i: (i + nblk, 0)),
                pl.BlockSpec((_BT, 1), lambda i: (i + nblk, 0)),
                pl.BlockSpec((1, d), lambda i: (0, 0)),
                pl.BlockSpec((1, d), lambda i: (0, 0)),
            ],
            out_specs=pl.BlockSpec((_BT, d), lambda i: (i + nblk, 0)),
            out_shape=jax.ShapeDtypeStruct((n, d), jnp.float32),
            input_output_aliases={0: 0},
        )(prev, rows_half, h2, active_f, ln_g.reshape(1, d),
          ln_b.reshape(1, d))

    idx_a, acc_a = s1(0)
    rows_a = gather(cbw, idx_a)
    idx_b, acc_b = s1(1)
    rows_b = gather(cbw, idx_b)
    hc_a = s3_first(rows_a)
    h_comm = s3_second(hc_a, rows_b)

    vq_loss = (1.0 + 0.25) * (acc_a[0, 0] + acc_b[0, 0]) / (n * d)
    return h_comm.reshape(hidden.shape), vq_loss
